# Initial kernel scaffold; baseline (speedup 1.0000x reference)
import jax
import jax.numpy as jnp
from jax.experimental import pallas as pl


def kernel(x_player, x_history, edge_index_teammate, edge_index_enemy, edge_index_rev_has_history, batch_player, emb_p1, emb_p2, emb_p3, emb_p4, emb_p5, emb_h0, emb_h3, W_enc_p, b_enc_p, W_enc_h, b_enc_h, Wl_0_team, Wr_0_team, b_0_team, Wl_0_enemy, Wr_0_enemy, b_0_enemy, Wl_0_hist, Wr_0_hist, b_0_hist, Wl_1_team, Wr_1_team, b_1_team, Wl_1_enemy, Wr_1_enemy, b_1_enemy, Wl_1_hist, Wr_1_hist, b_1_hist, W_cls, b_cls):
    raise NotImplementedError("write your pallas kernel here")


# speedup vs baseline: 1.0413x; 1.0413x over previous
import functools

import jax
import jax.numpy as jnp
from jax import lax
from jax.experimental import pallas as pl
from jax.experimental.pallas import tpu as pltpu
from jax.experimental.pallas import tpu_sc as plsc

N_P = 50000
N_H = 50000
N_GRAPHS = 5000
HID = 128

N_PAD = 50176
NG_PAD = 5120

NC, NS = 2, 16
CE = 128
ZB = 160

F32 = jnp.float32
I32 = jnp.int32


def _sds(shape, dtype):
    return jax.ShapeDtypeStruct(shape, dtype)


def _mesh():
    return plsc.VectorSubcoreMesh(core_axis_name="c", subcore_axis_name="s",
                                  num_cores=NC, num_subcores=NS)


def _sc_emb_sum(table, idx, n_rows, group):
    assert n_rows % CE == 0
    nch = n_rows // CE
    nrounds = (nch + NC * NS - 1) // (NC * NS)

    def body(tab_hbm, idx_hbm, out_hbm, idxv, rows, acc, sem):
        del sem
        c = lax.axis_index("c")
        s = lax.axis_index("s")
        wid = s * NC + c

        @pl.loop(0, nrounds)
        def _round(j):
            chunk = j * (NC * NS) + wid

            @pl.when(chunk < nch)
            def _do():
                for g in range(group):
                    pltpu.sync_copy(
                        idx_hbm.at[pl.ds(g * n_rows + chunk * CE, CE)], idxv)
                    pltpu.sync_copy(tab_hbm.at[idxv], rows)

                    @pl.loop(0, CE)
                    def _r(r):
                        for j8 in range(HID // 16):
                            sl = pl.ds(j8 * 16, 16)
                            if g == 0:
                                acc[r, sl] = rows[r, sl]
                            else:
                                acc[r, sl] = acc[r, sl] + rows[r, sl]
                pltpu.sync_copy(acc, out_hbm.at[pl.ds(chunk * CE, CE)])

    kfn = pl.kernel(
        body,
        out_type=_sds((n_rows, HID), F32),
        mesh=_mesh(),
        scratch_types=[
            pltpu.VMEM((CE,), I32),
            pltpu.VMEM((CE, HID), F32),
            pltpu.VMEM((CE, HID), F32),
            pltpu.SemaphoreType.DMA,
        ],
    )
    return kfn(table, idx.reshape(-1))


_DOT = functools.partial(jnp.dot, preferred_element_type=F32,
                         precision=lax.Precision.HIGHEST)
_BLK = 512


def _tc_encode(cont, wc, b, embsum):
    def body(c_ref, w_ref, b_ref, e_ref, o_ref):
        o_ref[...] = _DOT(c_ref[...], w_ref[...]) + b_ref[...] + e_ref[...]

    n = cont.shape[0]
    return pl.pallas_call(
        body,
        grid=(n // _BLK,),
        in_specs=[
            pl.BlockSpec((_BLK, 8), lambda i: (i, 0)),
            pl.BlockSpec((8, HID), lambda i: (0, 0)),
            pl.BlockSpec((1, HID), lambda i: (0, 0)),
            pl.BlockSpec((_BLK, HID), lambda i: (i, 0)),
        ],
        out_specs=pl.BlockSpec((_BLK, HID), lambda i: (i, 0)),
        out_shape=_sds((n, HID), F32),
    )(cont, wc, b, embsum)


def _tc_layer(aggs, cnts, xp, wls, wr_sum, b_sum):
    def body(a0, a1, a2, c0, c1, c2, x_ref, w0, w1, w2, wr, b_ref, o_ref):
        acc = _DOT(x_ref[...], wr[...]) + b_ref[...]
        for a, cn, w in ((a0, c0, w0), (a1, c1, w1), (a2, c2, w2)):
            mean = a[...] / jnp.maximum(cn[...], 1.0)
            acc = acc + _DOT(mean, w[...])
        o_ref[...] = jnp.maximum(acc, 0.0)

    n = xp.shape[0]
    row = pl.BlockSpec((_BLK, HID), lambda i: (i, 0))
    cntb = pl.BlockSpec((_BLK, 1), lambda i: (i, 0))
    wspec = pl.BlockSpec((HID, HID), lambda i: (0, 0))
    return pl.pallas_call(
        body,
        grid=(n // _BLK,),
        in_specs=[row, row, row, cntb, cntb, cntb, row,
                  wspec, wspec, wspec, wspec,
                  pl.BlockSpec((1, HID), lambda i: (0, 0))],
        out_specs=row,
        out_shape=_sds((n, HID), F32),
    )(*aggs, *cnts, xp, *wls, wr_sum, b_sum)


def _tc_classify(pooled, cnt, w_cls, b_cls):
    def body(p_ref, c_ref, w_ref, b_ref, o_ref):
        h = p_ref[...] / jnp.maximum(c_ref[...], 1.0)
        o_ref[...] = _DOT(h, w_ref[...]) + b_ref[...]

    n = pooled.shape[0]
    return pl.pallas_call(
        body,
        grid=(1,),
        in_specs=[
            pl.BlockSpec((n, HID), lambda i: (0, 0)),
            pl.BlockSpec((n, 1), lambda i: (0, 0)),
            pl.BlockSpec((HID, 1), lambda i: (0, 0)),
            pl.BlockSpec((1, 1), lambda i: (0, 0)),
        ],
        out_specs=pl.BlockSpec((n, 1), lambda i: (0, 0)),
        out_shape=_sds((n, 1), F32),
    )(pooled, cnt, w_cls, b_cls)


PLAYER_EMB_SPEC = [(1, 200), (2, 2000), (3, 50), (4, 10), (5, 170)]
HIST_EMB_SPEC = [(0, 2000), (3, 10)]
P_CONT = [0, 6, 7, 8, 9, 10, 11]
H_CONT = [1, 2, 4, 5, 6, 7]


def _pad_rows(x, n):
    return jnp.pad(x, ((0, n - x.shape[0]),) + ((0, 0),) * (x.ndim - 1))


def _prep_ids(x, spec, n_rows):
    cols, offs, off = [], [], 0
    for idx, vocab in spec:
        ids = jnp.clip(x[:, idx].astype(I32), 0, vocab - 1) + off
        cols.append(ids)
        offs.append(off)
        off += vocab
    ids = jnp.stack(cols, axis=0)
    return jnp.pad(ids, ((0, 0), (0, n_rows - ids.shape[1])))


def _prep_edges(ei, n_src):
    e = ei.shape[1]
    ep = ((e + CE * NS - 1) // (CE * NS)) * (CE * NS)
    src = jnp.pad(ei[0], (0, ep - e))
    dst = jnp.pad(ei[1], (0, ep - e), constant_values=jnp.int32(1 << 30))
    return src, dst


def kernel(x_player, x_history, edge_index_teammate, edge_index_enemy,
           edge_index_rev_has_history, batch_player,
           emb_p1, emb_p2, emb_p3, emb_p4, emb_p5, emb_h0, emb_h3,
           W_enc_p, b_enc_p, W_enc_h, b_enc_h,
           Wl_0_team, Wr_0_team, b_0_team,
           Wl_0_enemy, Wr_0_enemy, b_0_enemy,
           Wl_0_hist, Wr_0_hist, b_0_hist,
           Wl_1_team, Wr_1_team, b_1_team,
           Wl_1_enemy, Wr_1_enemy, b_1_enemy,
           Wl_1_hist, Wr_1_hist, b_1_hist,
           W_cls, b_cls):
    tabs_p, r = [], 7
    for tab in (emb_p1, emb_p2, emb_p3, emb_p4, emb_p5):
        d = tab.shape[1]
        tabs_p.append(tab @ W_enc_p[r:r + d])
        r += d
    table_p = jnp.concatenate(tabs_p, axis=0)
    tabs_h, r = [], 6
    for tab in (emb_h0, emb_h3):
        d = tab.shape[1]
        tabs_h.append(tab @ W_enc_h[r:r + d])
        r += d
    table_h = jnp.concatenate(tabs_h, axis=0)

    wc_p = jnp.pad(W_enc_p[:7], ((0, 1), (0, 0)))
    wc_h = jnp.pad(W_enc_h[:6], ((0, 2), (0, 0)))
    b_p = b_enc_p.reshape(1, HID)
    b_h = b_enc_h.reshape(1, HID)

    layer_w = []
    for wl_t, wl_e, wl_h, wr_t, wr_e, wr_h, bt, be, bh in (
            (Wl_0_team, Wl_0_enemy, Wl_0_hist,
             Wr_0_team, Wr_0_enemy, Wr_0_hist, b_0_team, b_0_enemy, b_0_hist),
            (Wl_1_team, Wl_1_enemy, Wl_1_hist,
             Wr_1_team, Wr_1_enemy, Wr_1_hist, b_1_team, b_1_enemy, b_1_hist)):
        layer_w.append(((wl_t, wl_e, wl_h), wr_t + wr_e + wr_h,
                        (bt + be + bh).reshape(1, HID)))

    ids_p = _prep_ids(x_player, PLAYER_EMB_SPEC, N_PAD)
    ids_h = _prep_ids(x_history, HIST_EMB_SPEC, N_PAD)
    cont_p = _pad_rows(jnp.pad(x_player[:, jnp.array(P_CONT)],
                               ((0, 0), (0, 1))), N_PAD)
    cont_h = _pad_rows(jnp.pad(x_history[:, jnp.array(H_CONT)],
                               ((0, 0), (0, 2))), N_PAD)

    rels = (("team", edge_index_teammate), ("enemy", edge_index_enemy),
            ("hist", edge_index_rev_has_history))
    srcs = {}
    for rel, ei in rels:
        e = ei.shape[1]
        ep = ((e + CE - 1) // CE) * CE
        srcs[rel] = jnp.pad(ei[0], (0, ep - e))
    cnts = [
        _pad_rows(jax.ops.segment_sum(jnp.ones((ei.shape[1], 1), F32), ei[1],
                                      num_segments=N_P), N_PAD)
        for _, ei in rels]

    emb_sum_p = _sc_emb_sum(table_p, ids_p, N_PAD, len(PLAYER_EMB_SPEC))
    emb_sum_h = _sc_emb_sum(table_h, ids_h, N_PAD, len(HIST_EMB_SPEC))
    xp = _tc_encode(cont_p, wc_p, b_p, emb_sum_p)
    xh = _tc_encode(cont_h, wc_h, b_h, emb_sum_h)

    for (wls, wr_sum, b_sum) in layer_w:
        aggs = []
        for rel, ei in rels:
            x_src = xh if rel == "hist" else xp
            msg = _sc_emb_sum(x_src, srcs[rel], srcs[rel].shape[0], 1)
            agg = jax.ops.segment_sum(msg[:ei.shape[1]], ei[1],
                                      num_segments=N_P)
            aggs.append(_pad_rows(agg, N_PAD))
        xp = _tc_layer(aggs, cnts, xp, wls, wr_sum, b_sum)

    xp_v = xp[:N_P]
    pooled = jax.ops.segment_sum(xp_v, batch_player, num_segments=N_GRAPHS)
    cnt_g = jax.ops.segment_sum(jnp.ones((N_P, 1), F32), batch_player,
                                num_segments=N_GRAPHS)
    return _tc_classify(pooled, cnt_g, W_cls, b_cls.reshape(1, 1))
